# baseline (speedup 1.0000x reference)
import functools

import jax
import jax.numpy as jnp
from jax import lax
from jax.experimental import pallas as pl
from jax.experimental.pallas import tpu as pltpu, tpu_sc as plsc

_T = 64
_P = _T - 1
_D = 64
_L = 16

_info = plsc.get_sparse_core_info()
_NC = _info.num_cores

_mesh = plsc.VectorSubcoreMesh(core_axis_name="c", subcore_axis_name="s")


@functools.partial(
    pl.kernel,
    out_type=jax.ShapeDtypeStruct((_T, _D), jnp.float32),
    mesh=_mesh,
    compiler_params=pltpu.CompilerParams(use_tc_tiling_on_sc=False),
    scratch_types=[
        pltpu.VMEM((_T + _L,), jnp.int32),
        pltpu.VMEM((_T,), jnp.int32),
        pltpu.VMEM((_T, _D), jnp.float32),
        pltpu.SemaphoreType.DMA,
    ],
)
def _sc_embed(base_hbm, table_hbm, out_hbm, base_v, dist_v, rows_v, sem):
    wid = lax.axis_index("s") * _NC + lax.axis_index("c")

    @pl.when(wid == 0)
    def _():
        pltpu.sync_copy(base_hbm, base_v.at[pl.ds(0, _T)])
        for c in range(_T // _L):
            prv = base_v[pl.ds(c * _L, _L)]
            nxt = base_v[pl.ds(c * _L + 1, _L)]
            d = jnp.minimum(jnp.maximum(nxt - prv, 0), 999)
            dist_v[pl.ds(c * _L, _L)] = d
        pltpu.async_copy(table_hbm.at[dist_v], rows_v, sem).wait()
        pltpu.sync_copy(rows_v, out_hbm)


def kernel(frame_index, embedding_table):
    base = frame_index.reshape(_T)
    rows = _sc_embed(base, embedding_table)
    return rows[:_P].reshape(_P, 1, 1, _D)


# speedup vs baseline: 1.0030x; 1.0030x over previous
import functools

import jax
import jax.numpy as jnp
from jax import lax
from jax.experimental import pallas as pl
from jax.experimental.pallas import tpu as pltpu, tpu_sc as plsc

_T = 64
_P = _T - 1
_D = 64
_L = 16

_info = plsc.get_sparse_core_info()
_NC = _info.num_cores

_mesh = plsc.VectorSubcoreMesh(core_axis_name="c", subcore_axis_name="s")


@functools.partial(
    pl.kernel,
    out_type=jax.ShapeDtypeStruct((_P, _D), jnp.float32),
    mesh=_mesh,
    compiler_params=pltpu.CompilerParams(use_tc_tiling_on_sc=False),
    scratch_types=[
        pltpu.VMEM((_T + _L,), jnp.int32),
        pltpu.VMEM((_T,), jnp.int32),
        pltpu.VMEM((_T, _D), jnp.float32),
        pltpu.SemaphoreType.DMA,
    ],
)
def _sc_embed(base_hbm, table_hbm, out_hbm, base_v, dist_v, rows_v, sem):
    wid = lax.axis_index("s") * _NC + lax.axis_index("c")

    @pl.when(wid == 0)
    def _():
        pltpu.sync_copy(base_hbm, base_v.at[pl.ds(0, _T)])
        for c in range(_T // _L):
            prv = base_v[pl.ds(c * _L, _L)]
            nxt = base_v[pl.ds(c * _L + 1, _L)]
            d = jnp.minimum(jnp.maximum(nxt - prv, 0), 999)
            dist_v[pl.ds(c * _L, _L)] = d
        pltpu.async_copy(table_hbm.at[dist_v], rows_v, sem).wait()
        pltpu.sync_copy(rows_v.at[pl.ds(0, _P)], out_hbm)


def kernel(frame_index, embedding_table):
    base = frame_index.reshape(_T)
    rows = _sc_embed(base, embedding_table)
    return rows.reshape(_P, 1, 1, _D)


# speedup vs baseline: 1.0631x; 1.0600x over previous
import functools

import jax
import jax.numpy as jnp
from jax import lax
from jax.experimental import pallas as pl
from jax.experimental.pallas import tpu as pltpu, tpu_sc as plsc

_T = 64
_P = _T - 1
_D = 64

_mesh = plsc.ScalarSubcoreMesh(axis_name="c")


@functools.partial(
    pl.kernel,
    out_type=jax.ShapeDtypeStruct((_P, _D), jnp.float32),
    mesh=_mesh,
    compiler_params=pltpu.CompilerParams(use_tc_tiling_on_sc=False),
    scratch_types=[
        pltpu.SMEM((_T,), jnp.int32),
        pltpu.SemaphoreType.DMA,
    ],
)
def _sc_embed(base_hbm, table_hbm, out_hbm, base_s, sem):
    @pl.when(lax.axis_index("c") == 0)
    def _():
        pltpu.sync_copy(base_hbm, base_s)
        descs = []
        for i in range(_P):
            d = base_s[i + 1] - base_s[i]
            descs.append(pltpu.async_copy(table_hbm.at[d], out_hbm.at[i], sem))
        for dsc in descs:
            dsc.wait()


def kernel(frame_index, embedding_table):
    base = frame_index.reshape(_T)
    rows = _sc_embed(base, embedding_table)
    return rows.reshape(_P, 1, 1, _D)


# speedup vs baseline: 1.1385x; 1.0710x over previous
import functools

import jax
import jax.numpy as jnp
from jax import lax
from jax.experimental import pallas as pl
from jax.experimental.pallas import tpu as pltpu, tpu_sc as plsc

_T = 64
_P = _T - 1
_D = 64

_mesh = plsc.ScalarSubcoreMesh(axis_name="c", num_cores=1)


@functools.partial(
    pl.kernel,
    out_type=jax.ShapeDtypeStruct((_P, _D), jnp.float32),
    mesh=_mesh,
    compiler_params=pltpu.CompilerParams(use_tc_tiling_on_sc=False),
    scratch_types=[
        pltpu.SMEM((_T,), jnp.int32),
        pltpu.SemaphoreType.DMA,
    ],
)
def _sc_embed(base_hbm, table_hbm, out_hbm, base_s, sem):
    @pl.when(lax.axis_index("c") == 0)
    def _():
        pltpu.sync_copy(base_hbm, base_s)
        descs = []
        for i in range(_P):
            d = base_s[i + 1] - base_s[i]
            descs.append(pltpu.async_copy(table_hbm.at[d], out_hbm.at[i], sem))
        for dsc in descs:
            dsc.wait()


def kernel(frame_index, embedding_table):
    base = frame_index.reshape(_T)
    rows = _sc_embed(base, embedding_table)
    return rows.reshape(_P, 1, 1, _D)


# speedup vs baseline: 1.1455x; 1.0061x over previous
import functools

import jax
import jax.numpy as jnp
from jax import lax
from jax.experimental import pallas as pl
from jax.experimental.pallas import tpu as pltpu, tpu_sc as plsc

_T = 64
_P = _T - 1
_D = 64

_mesh = plsc.ScalarSubcoreMesh(axis_name="c", num_cores=1)


@functools.partial(
    pl.kernel,
    out_type=jax.ShapeDtypeStruct((_P, _D), jnp.float32),
    mesh=_mesh,
    compiler_params=pltpu.CompilerParams(use_tc_tiling_on_sc=False),
    scratch_types=[
        pltpu.SMEM((_T,), jnp.int32),
        pltpu.SemaphoreType.DMA,
    ],
)
def _sc_embed(base_hbm, table_hbm, out_hbm, base_s, sem):
    @pl.when(lax.axis_index("c") == 0)
    def _():
        pltpu.sync_copy(base_hbm, base_s)
        for i in range(_P):
            d = base_s[i + 1] - base_s[i]
            pltpu.async_copy(table_hbm.at[d], out_hbm.at[i], sem)
        pltpu.make_async_copy(table_hbm.at[pl.ds(0, _P)], out_hbm, sem).wait()


def kernel(frame_index, embedding_table):
    base = frame_index.reshape(_T)
    rows = _sc_embed(base, embedding_table)
    return rows.reshape(_P, 1, 1, _D)
